# baseline (speedup 1.0000x reference)
import functools

import jax
import jax.numpy as jnp
from jax import lax
from jax.experimental import pallas as pl
from jax.experimental.pallas import tpu as pltpu
from jax.experimental.pallas import tpu_sc as plsc

N = 10000
NP = 10240
E = 160000
F = 64
NC = 2
NS = 16
C = 128
EPS_RAW = E // NS
NCH = -(-EPS_RAW // C)
NCH += (-NCH) % 4
EPS = NCH * C
E_PAD = EPS * NS


@functools.lru_cache(maxsize=None)
def _make_spmm(din):
    nslabs = din // F
    spc = nslabs // NC
    mesh = plsc.VectorSubcoreMesh(
        core_axis_name="c", subcore_axis_name="s", num_cores=NC, num_subcores=NS
    )
    grp = C // 16
    NB = 4

    @functools.partial(
        pl.kernel,
        mesh=mesh,
        compiler_params=pltpu.CompilerParams(use_tc_tiling_on_sc=False),
        out_type=jax.ShapeDtypeStruct((nslabs, NP, F), jnp.float32),
        scratch_types=[
            pltpu.VMEM((NCH, C), jnp.int32),
            pltpu.VMEM((C, F), jnp.float32),
            pltpu.VMEM((C, F), jnp.float32),
            pltpu.VMEM((C, F), jnp.float32),
            pltpu.VMEM((C, F), jnp.float32),
            pltpu.VMEM((NB, C), jnp.int32),
            pltpu.VMEM((NB, C), jnp.float32),
            pltpu.VMEM_SHARED((NP, F), jnp.float32),
            pltpu.SemaphoreType.DMA,
            pltpu.SemaphoreType.DMA,
            pltpu.SemaphoreType.DMA,
            pltpu.SemaphoreType.DMA,
            pltpu.SemaphoreType.DMA,
            pltpu.SemaphoreType.DMA,
            pltpu.SemaphoreType.DMA,
            pltpu.SemaphoreType.DMA,
        ],
    )
    def spmm(h2, colr, rowr, ewr, zeros, out, idx_v, rb0, rb1, rb2, rb3,
             row_cb, ew_cb, slab, sg0, sg1, sg2, sg3, ss0, ss1, ss2, ss3):
        cid = lax.axis_index("c")
        sid = lax.axis_index("s")
        stripe = pl.ds(sid * (NP // NS), NP // NS)
        rbufs = (rb0, rb1, rb2, rb3)
        sgs = (sg0, sg1, sg2, sg3)
        sss = (ss0, ss1, ss2, ss3)

        def issue(b, j):
            pltpu.async_copy(h2.at[idx_v.at[j]], rbufs[b], sgs[b])
            pltpu.async_copy(rowr.at[sid, j], row_cb.at[b], sgs[b])
            pltpu.async_copy(ewr.at[sid, j], ew_cb.at[b], sgs[b])

        def wait_in(b, j):
            pltpu.make_async_copy(h2.at[idx_v.at[j]], rbufs[b], sgs[b]).wait()
            pltpu.make_async_copy(rowr.at[sid, j], row_cb.at[b], sgs[b]).wait()
            pltpu.make_async_copy(ewr.at[sid, j], ew_cb.at[b], sgs[b]).wait()

        def scatter_start(b):
            pltpu.async_copy(rbufs[b], slab.at[row_cb.at[b]], sss[b], add=True)

        def scatter_wait(b):
            pltpu.make_async_copy(
                rbufs[b], slab.at[row_cb.at[b]], sss[b]).wait()

        for sl in range(spc):
            s = cid * spc + sl

            pltpu.sync_copy(zeros, slab.at[stripe, :])

            pltpu.sync_copy(colr.at[sid], idx_v)

            def idx_body(g, _):
                j = g // grp
                q = g - j * grp
                gs = pl.ds(q * 16, 16)
                idx_v[j, gs] = idx_v[j, gs] * nslabs + s
                return 0

            lax.fori_loop(0, NCH * grp, idx_body, 0)
            plsc.subcore_barrier()

            for b in range(NB - 1):
                issue(b, b)

            def quad_body(t, _):
                j0 = 4 * t
                for b in range(NB):
                    j = j0 + b
                    wait_in(b, j)

                    def scale_body(g, _):
                        w16 = ew_cb[b, pl.ds(g * 16, 16)]
                        for jj in range(16):
                            w = lax.broadcast(w16[jj], (16,))
                            e = g * 16 + jj
                            for f in range(F // 16):
                                fs = pl.ds(f * 16, 16)
                                rbufs[b][e, fs] = rbufs[b][e, fs] * w
                        return 0

                    lax.fori_loop(0, grp, scale_body, 0)
                    scatter_start(b)

                    p = (b + NB - 1) % NB
                    jp = j + NB - 1

                    @pl.when(jp >= NB)
                    def _():
                        scatter_wait(p)

                    @pl.when(jp < NCH)
                    def _():
                        issue(p, jp)

                return 0

            lax.fori_loop(0, NCH // 4, quad_body, 0)
            scatter_wait(NB - 1)
            plsc.subcore_barrier()

            pltpu.sync_copy(slab.at[stripe, :], out.at[s, stripe, :])

    return spmm


def _self_mm(h, w, b1, b2, bn=1000):
    n, din = h.shape
    hdim = w.shape[1]

    def body(h_ref, w_ref, b1_ref, b2_ref, o_ref):
        acc = jnp.dot(h_ref[...], w_ref[...], preferred_element_type=jnp.float32)
        o_ref[...] = acc + b1_ref[...] + b2_ref[...]

    return pl.pallas_call(
        body,
        grid=(n // bn,),
        in_specs=[
            pl.BlockSpec((bn, din), lambda i: (i, 0)),
            pl.BlockSpec((din, hdim), lambda i: (0, 0)),
            pl.BlockSpec((hdim,), lambda i: (0,)),
            pl.BlockSpec((hdim,), lambda i: (0,)),
        ],
        out_specs=pl.BlockSpec((bn, hdim), lambda i: (i, 0)),
        out_shape=jax.ShapeDtypeStruct((n, hdim), jnp.float32),
    )(h, w, b1, b2)


def _combine(hs, nei_t, wn_r, bn=1000):
    n, hdim = hs.shape
    nslabs = nei_t.shape[0]

    def body(hs_ref, nei_ref, wn_ref, o_ref):
        acc = hs_ref[...]
        for s in range(nslabs):
            acc = acc + jnp.dot(nei_ref[s], wn_ref[s],
                                preferred_element_type=jnp.float32)
        o_ref[...] = jnp.maximum(acc, 0.0)

    return pl.pallas_call(
        body,
        grid=(n // bn,),
        in_specs=[
            pl.BlockSpec((bn, hdim), lambda i: (i, 0)),
            pl.BlockSpec((nslabs, bn, F), lambda i: (0, i, 0)),
            pl.BlockSpec((nslabs, F, hdim), lambda i: (0, 0, 0)),
        ],
        out_specs=pl.BlockSpec((bn, hdim), lambda i: (i, 0)),
        out_shape=jax.ShapeDtypeStruct((n, hdim), jnp.float32),
    )(hs, nei_t, wn_r)


def _combine_final(hs, nei_t, wn_r, w_out, b_out, bn=1000):
    n, hdim = hs.shape
    nslabs = nei_t.shape[0]

    def body(hs_ref, nei_ref, wn_ref, wo_ref, bo_ref, o_ref):
        acc = hs_ref[...]
        for s in range(nslabs):
            acc = acc + jnp.dot(nei_ref[s], wn_ref[s],
                                preferred_element_type=jnp.float32)
        acc = jnp.maximum(acc, 0.0)
        o_ref[...] = jnp.dot(acc, wo_ref[...],
                             preferred_element_type=jnp.float32) + bo_ref[...]

    return pl.pallas_call(
        body,
        grid=(n // bn,),
        in_specs=[
            pl.BlockSpec((bn, hdim), lambda i: (i, 0)),
            pl.BlockSpec((nslabs, bn, F), lambda i: (0, i, 0)),
            pl.BlockSpec((nslabs, F, hdim), lambda i: (0, 0, 0)),
            pl.BlockSpec((hdim, 1), lambda i: (0, 0)),
            pl.BlockSpec((1,), lambda i: (0,)),
        ],
        out_specs=pl.BlockSpec((bn, 1), lambda i: (i, 0)),
        out_shape=jax.ShapeDtypeStruct((n, 1), jnp.float32),
    )(hs, nei_t, wn_r, w_out, b_out)


def kernel(x, edge_index, edge_weight, W_self_0, b_self_0, W_nei_0, b_nei_0,
           W_self_1, b_self_1, W_nei_1, b_nei_1, W_self_2, b_self_2, W_nei_2,
           b_nei_2, W_out, b_out):
    row = edge_index[0]
    col = edge_index[1]
    pad = E_PAD - E
    col_r = jnp.pad(col, (0, pad)).reshape(NS, NCH, C)
    row_r = jnp.pad(row, (0, pad)).reshape(NS, NCH, C)
    ew_r = jnp.pad(edge_weight, (0, pad)).reshape(NS, NCH, C)
    zeros = jnp.zeros((NP // NS, F), jnp.float32)

    params = [
        (W_self_0, b_self_0, W_nei_0, b_nei_0),
        (W_self_1, b_self_1, W_nei_1, b_nei_1),
        (W_self_2, b_self_2, W_nei_2, b_nei_2),
    ]
    h = x
    for k, (ws, bs, wn, bnei) in enumerate(params):
        din = h.shape[1]
        nslabs = din // F
        h2 = h.reshape(N * nslabs, F)
        nei_t = _make_spmm(din)(h2, col_r, row_r, ew_r, zeros)
        hs = _self_mm(h, ws, bs, bnei)
        wn_r = wn.reshape(nslabs, F, wn.shape[1])
        if k < 2:
            h = _combine(hs, nei_t, wn_r)
        else:
            out = _combine_final(hs, nei_t, wn_r, W_out, b_out)
    return out[:, 0]


# speedup vs baseline: 1.1404x; 1.1404x over previous
import functools

import jax
import jax.numpy as jnp
from jax import lax
from jax.experimental import pallas as pl
from jax.experimental.pallas import tpu as pltpu
from jax.experimental.pallas import tpu_sc as plsc

N = 10000
NP = 10240
E = 160000
F = 128
NC = 2
NS = 16
C = 64
EPS_RAW = E // NS
NCH = -(-EPS_RAW // C)
NCH += (-NCH) % 4
EPS = NCH * C
E_PAD = EPS * NS


@functools.lru_cache(maxsize=None)
def _make_spmm(din):
    nslabs = din // F
    spc = nslabs // NC
    mesh = plsc.VectorSubcoreMesh(
        core_axis_name="c", subcore_axis_name="s", num_cores=NC, num_subcores=NS
    )
    grp = C // 16
    NB = 2

    @functools.partial(
        pl.kernel,
        mesh=mesh,
        compiler_params=pltpu.CompilerParams(use_tc_tiling_on_sc=False),
        out_type=jax.ShapeDtypeStruct((nslabs, NP, F), jnp.float32),
        scratch_types=[
            pltpu.VMEM((NCH, C), jnp.int32),
            pltpu.VMEM((C, F), jnp.float32),
            pltpu.VMEM((C, F), jnp.float32),
            pltpu.VMEM((NB, C), jnp.int32),
            pltpu.VMEM((NB, C), jnp.float32),
            pltpu.VMEM_SHARED((NP, F), jnp.float32),
            pltpu.SemaphoreType.DMA,
            pltpu.SemaphoreType.DMA,
            pltpu.SemaphoreType.DMA,
            pltpu.SemaphoreType.DMA,
        ],
    )
    def spmm(h2, colr, rowr, ewr, zeros, out, idx_v, rb0, rb1,
             row_cb, ew_cb, slab, sg0, sg1, ss0, ss1):
        cid = lax.axis_index("c")
        sid = lax.axis_index("s")
        stripe = pl.ds(sid * (NP // NS), NP // NS)
        rbufs = (rb0, rb1)
        sgs = (sg0, sg1)
        sss = (ss0, ss1)

        def issue(b, j):
            pltpu.async_copy(h2.at[idx_v.at[j]], rbufs[b], sgs[b])
            pltpu.async_copy(rowr.at[sid, j], row_cb.at[b], sgs[b])
            pltpu.async_copy(ewr.at[sid, j], ew_cb.at[b], sgs[b])

        def wait_in(b, j):
            pltpu.make_async_copy(h2.at[idx_v.at[j]], rbufs[b], sgs[b]).wait()
            pltpu.make_async_copy(rowr.at[sid, j], row_cb.at[b], sgs[b]).wait()
            pltpu.make_async_copy(ewr.at[sid, j], ew_cb.at[b], sgs[b]).wait()

        def scatter_start(b):
            pltpu.async_copy(rbufs[b], slab.at[row_cb.at[b]], sss[b], add=True)

        def scatter_wait(b):
            pltpu.make_async_copy(
                rbufs[b], slab.at[row_cb.at[b]], sss[b]).wait()

        for sl in range(spc):
            s = cid * spc + sl

            pltpu.sync_copy(zeros, slab.at[stripe, :])

            pltpu.sync_copy(colr.at[sid], idx_v)

            def idx_body(g, _):
                j = g // grp
                q = g - j * grp
                gs = pl.ds(q * 16, 16)
                idx_v[j, gs] = idx_v[j, gs] * nslabs + s
                return 0

            lax.fori_loop(0, NCH * grp, idx_body, 0)
            plsc.subcore_barrier()

            for b in range(NB - 1):
                issue(b, b)

            def quad_body(t, _):
                j0 = NB * t
                for b in range(NB):
                    j = j0 + b
                    wait_in(b, j)

                    def scale_body(g, _):
                        w16 = ew_cb[b, pl.ds(g * 16, 16)]
                        for jj in range(16):
                            w = lax.broadcast(w16[jj], (16,))
                            e = g * 16 + jj
                            for f in range(F // 16):
                                fs = pl.ds(f * 16, 16)
                                rbufs[b][e, fs] = rbufs[b][e, fs] * w
                        return 0

                    lax.fori_loop(0, grp, scale_body, 0)
                    scatter_start(b)

                    p = (b + NB - 1) % NB
                    jp = j + NB - 1

                    @pl.when(jp >= NB)
                    def _():
                        scatter_wait(p)

                    @pl.when(jp < NCH)
                    def _():
                        issue(p, jp)

                return 0

            lax.fori_loop(0, NCH // NB, quad_body, 0)
            scatter_wait(NB - 1)
            plsc.subcore_barrier()

            pltpu.sync_copy(slab.at[stripe, :], out.at[s, stripe, :])

    return spmm


def _self_mm(h, w, b1, b2, bn=1000):
    n, din = h.shape
    hdim = w.shape[1]

    def body(h_ref, w_ref, b1_ref, b2_ref, o_ref):
        acc = jnp.dot(h_ref[...], w_ref[...], preferred_element_type=jnp.float32)
        o_ref[...] = acc + b1_ref[...] + b2_ref[...]

    return pl.pallas_call(
        body,
        grid=(n // bn,),
        in_specs=[
            pl.BlockSpec((bn, din), lambda i: (i, 0)),
            pl.BlockSpec((din, hdim), lambda i: (0, 0)),
            pl.BlockSpec((hdim,), lambda i: (0,)),
            pl.BlockSpec((hdim,), lambda i: (0,)),
        ],
        out_specs=pl.BlockSpec((bn, hdim), lambda i: (i, 0)),
        out_shape=jax.ShapeDtypeStruct((n, hdim), jnp.float32),
    )(h, w, b1, b2)


def _combine(hs, nei_t, wn_r, bn=1000):
    n, hdim = hs.shape
    nslabs = nei_t.shape[0]

    def body(hs_ref, nei_ref, wn_ref, o_ref):
        acc = hs_ref[...]
        for s in range(nslabs):
            acc = acc + jnp.dot(nei_ref[s], wn_ref[s],
                                preferred_element_type=jnp.float32)
        o_ref[...] = jnp.maximum(acc, 0.0)

    return pl.pallas_call(
        body,
        grid=(n // bn,),
        in_specs=[
            pl.BlockSpec((bn, hdim), lambda i: (i, 0)),
            pl.BlockSpec((nslabs, bn, F), lambda i: (0, i, 0)),
            pl.BlockSpec((nslabs, F, hdim), lambda i: (0, 0, 0)),
        ],
        out_specs=pl.BlockSpec((bn, hdim), lambda i: (i, 0)),
        out_shape=jax.ShapeDtypeStruct((n, hdim), jnp.float32),
    )(hs, nei_t, wn_r)


def _combine_final(hs, nei_t, wn_r, w_out, b_out, bn=1000):
    n, hdim = hs.shape
    nslabs = nei_t.shape[0]

    def body(hs_ref, nei_ref, wn_ref, wo_ref, bo_ref, o_ref):
        acc = hs_ref[...]
        for s in range(nslabs):
            acc = acc + jnp.dot(nei_ref[s], wn_ref[s],
                                preferred_element_type=jnp.float32)
        acc = jnp.maximum(acc, 0.0)
        o_ref[...] = jnp.dot(acc, wo_ref[...],
                             preferred_element_type=jnp.float32) + bo_ref[...]

    return pl.pallas_call(
        body,
        grid=(n // bn,),
        in_specs=[
            pl.BlockSpec((bn, hdim), lambda i: (i, 0)),
            pl.BlockSpec((nslabs, bn, F), lambda i: (0, i, 0)),
            pl.BlockSpec((nslabs, F, hdim), lambda i: (0, 0, 0)),
            pl.BlockSpec((hdim, 1), lambda i: (0, 0)),
            pl.BlockSpec((1,), lambda i: (0,)),
        ],
        out_specs=pl.BlockSpec((bn, 1), lambda i: (i, 0)),
        out_shape=jax.ShapeDtypeStruct((n, 1), jnp.float32),
    )(hs, nei_t, wn_r, w_out, b_out)


def kernel(x, edge_index, edge_weight, W_self_0, b_self_0, W_nei_0, b_nei_0,
           W_self_1, b_self_1, W_nei_1, b_nei_1, W_self_2, b_self_2, W_nei_2,
           b_nei_2, W_out, b_out):
    row = edge_index[0]
    col = edge_index[1]
    pad = E_PAD - E
    col_r = jnp.pad(col, (0, pad)).reshape(NS, NCH, C)
    row_r = jnp.pad(row, (0, pad)).reshape(NS, NCH, C)
    ew_r = jnp.pad(edge_weight, (0, pad)).reshape(NS, NCH, C)
    zeros = jnp.zeros((NP // NS, F), jnp.float32)

    params = [
        (W_self_0, b_self_0, W_nei_0, b_nei_0),
        (W_self_1, b_self_1, W_nei_1, b_nei_1),
        (W_self_2, b_self_2, W_nei_2, b_nei_2),
    ]
    h = x
    for k, (ws, bs, wn, bnei) in enumerate(params):
        din = h.shape[1]
        nslabs = din // F
        h2 = h.reshape(N * nslabs, F)
        nei_t = _make_spmm(din)(h2, col_r, row_r, ew_r, zeros)
        hs = _self_mm(h, ws, bs, bnei)
        wn_r = wn.reshape(nslabs, F, wn.shape[1])
        if k < 2:
            h = _combine(hs, nei_t, wn_r)
        else:
            out = _combine_final(hs, nei_t, wn_r, W_out, b_out)
    return out[:, 0]


# speedup vs baseline: 2.0577x; 1.8044x over previous
import functools

import jax
import jax.numpy as jnp
from jax import lax
from jax.experimental import pallas as pl
from jax.experimental.pallas import tpu as pltpu
from jax.experimental.pallas import tpu_sc as plsc

N = 10000
NP = 10240
E = 160000
F = 128
NC = 2
NS = 16
C = 32
EPS_RAW = E // NS
NCH = -(-EPS_RAW // C)
NCH += (-NCH) % 4
EPS = NCH * C
E_PAD = EPS * NS


@functools.lru_cache(maxsize=None)
def _make_spmm(din):
    nslabs = din // F
    spc = nslabs // NC
    mesh = plsc.VectorSubcoreMesh(
        core_axis_name="c", subcore_axis_name="s", num_cores=NC, num_subcores=NS
    )
    grp = C // 16
    NB = 4

    @functools.partial(
        pl.kernel,
        mesh=mesh,
        compiler_params=pltpu.CompilerParams(use_tc_tiling_on_sc=False),
        out_type=jax.ShapeDtypeStruct((nslabs, NP, F), jnp.float32),
        scratch_types=[
            pltpu.VMEM((NCH, C), jnp.int32),
            pltpu.VMEM((C, F), jnp.float32),
            pltpu.VMEM((C, F), jnp.float32),
            pltpu.VMEM((C, F), jnp.float32),
            pltpu.VMEM((C, F), jnp.float32),
            pltpu.VMEM((NB, C), jnp.int32),
            pltpu.VMEM((NB, C), jnp.float32),
            pltpu.VMEM_SHARED((NP, F), jnp.float32),
            pltpu.SemaphoreType.DMA,
            pltpu.SemaphoreType.DMA,
            pltpu.SemaphoreType.DMA,
            pltpu.SemaphoreType.DMA,
            pltpu.SemaphoreType.DMA,
            pltpu.SemaphoreType.DMA,
            pltpu.SemaphoreType.DMA,
            pltpu.SemaphoreType.DMA,
        ],
    )
    def spmm(h2, colr, rowr, ewr, zeros, out, idx_v, rb0, rb1, rb2, rb3,
             row_cb, ew_cb, slab, sg0, sg1, sg2, sg3, ss0, ss1, ss2, ss3):
        cid = lax.axis_index("c")
        sid = lax.axis_index("s")
        stripe = pl.ds(sid * (NP // NS), NP // NS)
        rbufs = (rb0, rb1, rb2, rb3)
        sgs = (sg0, sg1, sg2, sg3)
        sss = (ss0, ss1, ss2, ss3)

        def issue(b, j):
            pltpu.async_copy(h2.at[idx_v.at[j]], rbufs[b], sgs[b])
            pltpu.async_copy(rowr.at[sid, j], row_cb.at[b], sgs[b])
            pltpu.async_copy(ewr.at[sid, j], ew_cb.at[b], sgs[b])

        def wait_in(b, j):
            pltpu.make_async_copy(h2.at[idx_v.at[j]], rbufs[b], sgs[b]).wait()
            pltpu.make_async_copy(rowr.at[sid, j], row_cb.at[b], sgs[b]).wait()
            pltpu.make_async_copy(ewr.at[sid, j], ew_cb.at[b], sgs[b]).wait()

        def scatter_start(b):
            pltpu.async_copy(rbufs[b], slab.at[row_cb.at[b]], sss[b], add=True)

        def scatter_wait(b):
            pltpu.make_async_copy(
                rbufs[b], slab.at[row_cb.at[b]], sss[b]).wait()

        for sl in range(spc):
            s = cid * spc + sl

            pltpu.sync_copy(zeros, slab.at[stripe, :])

            pltpu.sync_copy(colr.at[sid], idx_v)

            def idx_body(g, _):
                j = g // grp
                q = g - j * grp
                gs = pl.ds(q * 16, 16)
                idx_v[j, gs] = idx_v[j, gs] * nslabs + s
                return 0

            lax.fori_loop(0, NCH * grp, idx_body, 0)
            plsc.subcore_barrier()

            for b in range(NB - 1):
                issue(b, b)

            def quad_body(t, _):
                j0 = 4 * t
                for b in range(NB):
                    j = j0 + b
                    wait_in(b, j)

                    def scale_body(g, _):
                        w16 = ew_cb[b, pl.ds(g * 16, 16)]
                        for jj in range(16):
                            w = lax.broadcast(w16[jj], (16,))
                            e = g * 16 + jj
                            for f in range(F // 16):
                                fs = pl.ds(f * 16, 16)
                                rbufs[b][e, fs] = rbufs[b][e, fs] * w
                        return 0

                    lax.fori_loop(0, grp, scale_body, 0)
                    scatter_start(b)

                    p = (b + NB - 1) % NB
                    jp = j + NB - 1

                    @pl.when(jp >= NB)
                    def _():
                        scatter_wait(p)

                    @pl.when(jp < NCH)
                    def _():
                        issue(p, jp)

                return 0

            lax.fori_loop(0, NCH // 4, quad_body, 0)
            scatter_wait(NB - 1)
            plsc.subcore_barrier()

            pltpu.sync_copy(slab.at[stripe, :], out.at[s, stripe, :])

    return spmm


def _self_mm(h, w, b1, b2, bn=1000):
    n, din = h.shape
    hdim = w.shape[1]

    def body(h_ref, w_ref, b1_ref, b2_ref, o_ref):
        acc = jnp.dot(h_ref[...], w_ref[...], preferred_element_type=jnp.float32)
        o_ref[...] = acc + b1_ref[...] + b2_ref[...]

    return pl.pallas_call(
        body,
        grid=(n // bn,),
        in_specs=[
            pl.BlockSpec((bn, din), lambda i: (i, 0)),
            pl.BlockSpec((din, hdim), lambda i: (0, 0)),
            pl.BlockSpec((hdim,), lambda i: (0,)),
            pl.BlockSpec((hdim,), lambda i: (0,)),
        ],
        out_specs=pl.BlockSpec((bn, hdim), lambda i: (i, 0)),
        out_shape=jax.ShapeDtypeStruct((n, hdim), jnp.float32),
    )(h, w, b1, b2)


def _combine(hs, nei_t, wn_r, bn=1000):
    n, hdim = hs.shape
    nslabs = nei_t.shape[0]

    def body(hs_ref, nei_ref, wn_ref, o_ref):
        acc = hs_ref[...]
        for s in range(nslabs):
            acc = acc + jnp.dot(nei_ref[s], wn_ref[s],
                                preferred_element_type=jnp.float32)
        o_ref[...] = jnp.maximum(acc, 0.0)

    return pl.pallas_call(
        body,
        grid=(n // bn,),
        in_specs=[
            pl.BlockSpec((bn, hdim), lambda i: (i, 0)),
            pl.BlockSpec((nslabs, bn, F), lambda i: (0, i, 0)),
            pl.BlockSpec((nslabs, F, hdim), lambda i: (0, 0, 0)),
        ],
        out_specs=pl.BlockSpec((bn, hdim), lambda i: (i, 0)),
        out_shape=jax.ShapeDtypeStruct((n, hdim), jnp.float32),
    )(hs, nei_t, wn_r)


def _combine_final(hs, nei_t, wn_r, w_out, b_out, bn=1000):
    n, hdim = hs.shape
    nslabs = nei_t.shape[0]

    def body(hs_ref, nei_ref, wn_ref, wo_ref, bo_ref, o_ref):
        acc = hs_ref[...]
        for s in range(nslabs):
            acc = acc + jnp.dot(nei_ref[s], wn_ref[s],
                                preferred_element_type=jnp.float32)
        acc = jnp.maximum(acc, 0.0)
        o_ref[...] = jnp.dot(acc, wo_ref[...],
                             preferred_element_type=jnp.float32) + bo_ref[...]

    return pl.pallas_call(
        body,
        grid=(n // bn,),
        in_specs=[
            pl.BlockSpec((bn, hdim), lambda i: (i, 0)),
            pl.BlockSpec((nslabs, bn, F), lambda i: (0, i, 0)),
            pl.BlockSpec((nslabs, F, hdim), lambda i: (0, 0, 0)),
            pl.BlockSpec((hdim, 1), lambda i: (0, 0)),
            pl.BlockSpec((1,), lambda i: (0,)),
        ],
        out_specs=pl.BlockSpec((bn, 1), lambda i: (i, 0)),
        out_shape=jax.ShapeDtypeStruct((n, 1), jnp.float32),
    )(hs, nei_t, wn_r, w_out, b_out)


def kernel(x, edge_index, edge_weight, W_self_0, b_self_0, W_nei_0, b_nei_0,
           W_self_1, b_self_1, W_nei_1, b_nei_1, W_self_2, b_self_2, W_nei_2,
           b_nei_2, W_out, b_out):
    row = edge_index[0]
    col = edge_index[1]
    pad = E_PAD - E
    col_r = jnp.pad(col, (0, pad)).reshape(NS, NCH, C)
    row_r = jnp.pad(row, (0, pad)).reshape(NS, NCH, C)
    ew_r = jnp.pad(edge_weight, (0, pad)).reshape(NS, NCH, C)
    zeros = jnp.zeros((NP // NS, F), jnp.float32)

    params = [
        (W_self_0, b_self_0, W_nei_0, b_nei_0),
        (W_self_1, b_self_1, W_nei_1, b_nei_1),
        (W_self_2, b_self_2, W_nei_2, b_nei_2),
    ]
    h = x
    for k, (ws, bs, wn, bnei) in enumerate(params):
        din = h.shape[1]
        nslabs = din // F
        h2 = h.reshape(N * nslabs, F)
        nei_t = _make_spmm(din)(h2, col_r, row_r, ew_r, zeros)
        hs = _self_mm(h, ws, bs, bnei)
        wn_r = wn.reshape(nslabs, F, wn.shape[1])
        if k < 2:
            h = _combine(hs, nei_t, wn_r)
        else:
            out = _combine_final(hs, nei_t, wn_r, W_out, b_out)
    return out[:, 0]


# speedup vs baseline: 2.0583x; 1.0003x over previous
import functools

import jax
import jax.numpy as jnp
from jax import lax
from jax.experimental import pallas as pl
from jax.experimental.pallas import tpu as pltpu
from jax.experimental.pallas import tpu_sc as plsc

N = 10000
NP = 10240
E = 160000
F = 128
NC = 2
NS = 16
C = 32
EPS_RAW = E // NS
NCH = -(-EPS_RAW // C)
NCH += (-NCH) % 4
EPS = NCH * C
E_PAD = EPS * NS


@functools.lru_cache(maxsize=None)
def _make_spmm(din):
    nslabs = din // F
    spc = nslabs // NC
    mesh = plsc.VectorSubcoreMesh(
        core_axis_name="c", subcore_axis_name="s", num_cores=NC, num_subcores=NS
    )
    grp = C // 16
    NB = 4

    @functools.partial(
        pl.kernel,
        mesh=mesh,
        compiler_params=pltpu.CompilerParams(use_tc_tiling_on_sc=False),
        out_type=jax.ShapeDtypeStruct((nslabs, NP, F), jnp.float32),
        scratch_types=[
            pltpu.VMEM((NCH, C), jnp.int32),
            pltpu.VMEM((C, F), jnp.float32),
            pltpu.VMEM((C, F), jnp.float32),
            pltpu.VMEM((C, F), jnp.float32),
            pltpu.VMEM((C, F), jnp.float32),
            pltpu.VMEM((NB, C), jnp.int32),
            pltpu.VMEM((NB, C), jnp.float32),
            pltpu.VMEM_SHARED((NP, F), jnp.float32),
            pltpu.SemaphoreType.DMA,
            pltpu.SemaphoreType.DMA,
            pltpu.SemaphoreType.DMA,
            pltpu.SemaphoreType.DMA,
            pltpu.SemaphoreType.DMA,
            pltpu.SemaphoreType.DMA,
            pltpu.SemaphoreType.DMA,
            pltpu.SemaphoreType.DMA,
        ],
    )
    def spmm(h2, colr, rowr, ewr, zeros, out, idx_v, rb0, rb1, rb2, rb3,
             row_cb, ew_cb, slab, sg0, sg1, sg2, sg3, ss0, ss1, ss2, ss3):
        cid = lax.axis_index("c")
        sid = lax.axis_index("s")
        stripe = pl.ds(sid * (NP // NS), NP // NS)
        rbufs = (rb0, rb1, rb2, rb3)
        sgs = (sg0, sg1, sg2, sg3)
        sss = (ss0, ss1, ss2, ss3)

        NSPLIT = 2
        CS = C // NSPLIT

        def issue(b, j):
            for hsp in range(NSPLIT):
                sl = pl.ds(hsp * CS, CS)
                pltpu.async_copy(h2.at[idx_v.at[j, sl]], rbufs[b].at[sl, :],
                                 sgs[b])
            pltpu.async_copy(rowr.at[sid, j], row_cb.at[b], sgs[b])
            pltpu.async_copy(ewr.at[sid, j], ew_cb.at[b], sgs[b])

        def wait_in(b, j):
            for hsp in range(NSPLIT):
                sl = pl.ds(hsp * CS, CS)
                pltpu.make_async_copy(h2.at[idx_v.at[j, sl]],
                                      rbufs[b].at[sl, :], sgs[b]).wait()
            pltpu.make_async_copy(rowr.at[sid, j], row_cb.at[b], sgs[b]).wait()
            pltpu.make_async_copy(ewr.at[sid, j], ew_cb.at[b], sgs[b]).wait()

        def scatter_start(b):
            pltpu.async_copy(rbufs[b], slab.at[row_cb.at[b]], sss[b], add=True)

        def scatter_wait(b):
            pltpu.make_async_copy(
                rbufs[b], slab.at[row_cb.at[b]], sss[b]).wait()

        for sl in range(spc):
            s = cid * spc + sl

            pltpu.sync_copy(zeros, slab.at[stripe, :])

            pltpu.sync_copy(colr.at[sid], idx_v)

            def idx_body(g, _):
                j = g // grp
                q = g - j * grp
                gs = pl.ds(q * 16, 16)
                idx_v[j, gs] = idx_v[j, gs] * nslabs + s
                return 0

            lax.fori_loop(0, NCH * grp, idx_body, 0)
            plsc.subcore_barrier()

            for b in range(NB - 1):
                issue(b, b)

            def quad_body(t, _):
                j0 = 4 * t
                for b in range(NB):
                    j = j0 + b
                    wait_in(b, j)

                    def scale_body(g, _):
                        w16 = ew_cb[b, pl.ds(g * 16, 16)]
                        for jj in range(16):
                            w = lax.broadcast(w16[jj], (16,))
                            e = g * 16 + jj
                            for f in range(F // 16):
                                fs = pl.ds(f * 16, 16)
                                rbufs[b][e, fs] = rbufs[b][e, fs] * w
                        return 0

                    lax.fori_loop(0, grp, scale_body, 0)
                    scatter_start(b)

                    p = (b + NB - 1) % NB
                    jp = j + NB - 1

                    @pl.when(jp >= NB)
                    def _():
                        scatter_wait(p)

                    @pl.when(jp < NCH)
                    def _():
                        issue(p, jp)

                return 0

            lax.fori_loop(0, NCH // 4, quad_body, 0)
            scatter_wait(NB - 1)
            plsc.subcore_barrier()

            pltpu.sync_copy(slab.at[stripe, :], out.at[s, stripe, :])

    return spmm


def _self_mm(h, w, b1, b2, bn=1000):
    n, din = h.shape
    hdim = w.shape[1]

    def body(h_ref, w_ref, b1_ref, b2_ref, o_ref):
        acc = jnp.dot(h_ref[...], w_ref[...], preferred_element_type=jnp.float32)
        o_ref[...] = acc + b1_ref[...] + b2_ref[...]

    return pl.pallas_call(
        body,
        grid=(n // bn,),
        in_specs=[
            pl.BlockSpec((bn, din), lambda i: (i, 0)),
            pl.BlockSpec((din, hdim), lambda i: (0, 0)),
            pl.BlockSpec((hdim,), lambda i: (0,)),
            pl.BlockSpec((hdim,), lambda i: (0,)),
        ],
        out_specs=pl.BlockSpec((bn, hdim), lambda i: (i, 0)),
        out_shape=jax.ShapeDtypeStruct((n, hdim), jnp.float32),
    )(h, w, b1, b2)


def _combine(hs, nei_t, wn_r, bn=1000):
    n, hdim = hs.shape
    nslabs = nei_t.shape[0]

    def body(hs_ref, nei_ref, wn_ref, o_ref):
        acc = hs_ref[...]
        for s in range(nslabs):
            acc = acc + jnp.dot(nei_ref[s], wn_ref[s],
                                preferred_element_type=jnp.float32)
        o_ref[...] = jnp.maximum(acc, 0.0)

    return pl.pallas_call(
        body,
        grid=(n // bn,),
        in_specs=[
            pl.BlockSpec((bn, hdim), lambda i: (i, 0)),
            pl.BlockSpec((nslabs, bn, F), lambda i: (0, i, 0)),
            pl.BlockSpec((nslabs, F, hdim), lambda i: (0, 0, 0)),
        ],
        out_specs=pl.BlockSpec((bn, hdim), lambda i: (i, 0)),
        out_shape=jax.ShapeDtypeStruct((n, hdim), jnp.float32),
    )(hs, nei_t, wn_r)


def _combine_final(hs, nei_t, wn_r, w_out, b_out, bn=1000):
    n, hdim = hs.shape
    nslabs = nei_t.shape[0]

    def body(hs_ref, nei_ref, wn_ref, wo_ref, bo_ref, o_ref):
        acc = hs_ref[...]
        for s in range(nslabs):
            acc = acc + jnp.dot(nei_ref[s], wn_ref[s],
                                preferred_element_type=jnp.float32)
        acc = jnp.maximum(acc, 0.0)
        o_ref[...] = jnp.dot(acc, wo_ref[...],
                             preferred_element_type=jnp.float32) + bo_ref[...]

    return pl.pallas_call(
        body,
        grid=(n // bn,),
        in_specs=[
            pl.BlockSpec((bn, hdim), lambda i: (i, 0)),
            pl.BlockSpec((nslabs, bn, F), lambda i: (0, i, 0)),
            pl.BlockSpec((nslabs, F, hdim), lambda i: (0, 0, 0)),
            pl.BlockSpec((hdim, 1), lambda i: (0, 0)),
            pl.BlockSpec((1,), lambda i: (0,)),
        ],
        out_specs=pl.BlockSpec((bn, 1), lambda i: (i, 0)),
        out_shape=jax.ShapeDtypeStruct((n, 1), jnp.float32),
    )(hs, nei_t, wn_r, w_out, b_out)


def kernel(x, edge_index, edge_weight, W_self_0, b_self_0, W_nei_0, b_nei_0,
           W_self_1, b_self_1, W_nei_1, b_nei_1, W_self_2, b_self_2, W_nei_2,
           b_nei_2, W_out, b_out):
    row = edge_index[0]
    col = edge_index[1]
    pad = E_PAD - E
    col_r = jnp.pad(col, (0, pad)).reshape(NS, NCH, C)
    row_r = jnp.pad(row, (0, pad)).reshape(NS, NCH, C)
    ew_r = jnp.pad(edge_weight, (0, pad)).reshape(NS, NCH, C)
    zeros = jnp.zeros((NP // NS, F), jnp.float32)

    params = [
        (W_self_0, b_self_0, W_nei_0, b_nei_0),
        (W_self_1, b_self_1, W_nei_1, b_nei_1),
        (W_self_2, b_self_2, W_nei_2, b_nei_2),
    ]
    h = x
    for k, (ws, bs, wn, bnei) in enumerate(params):
        din = h.shape[1]
        nslabs = din // F
        h2 = h.reshape(N * nslabs, F)
        nei_t = _make_spmm(din)(h2, col_r, row_r, ew_r, zeros)
        hs = _self_mm(h, ws, bs, bnei)
        wn_r = wn.reshape(nslabs, F, wn.shape[1])
        if k < 2:
            h = _combine(hs, nei_t, wn_r)
        else:
            out = _combine_final(hs, nei_t, wn_r, W_out, b_out)
    return out[:, 0]


# speedup vs baseline: 2.0585x; 1.0001x over previous
import functools

import jax
import jax.numpy as jnp
from jax import lax
from jax.experimental import pallas as pl
from jax.experimental.pallas import tpu as pltpu
from jax.experimental.pallas import tpu_sc as plsc

N = 10000
NP = 10240
E = 160000
F = 128
NC = 2
NS = 16
C = 32
EPS_RAW = E // NS
NCH = -(-EPS_RAW // C)
NCH += (-NCH) % 4
EPS = NCH * C
E_PAD = EPS * NS


@functools.lru_cache(maxsize=None)
def _make_spmm(din):
    nslabs = din // F
    spc = nslabs // NC
    mesh = plsc.VectorSubcoreMesh(
        core_axis_name="c", subcore_axis_name="s", num_cores=NC, num_subcores=NS
    )
    grp = C // 16
    NB = 4

    @functools.partial(
        pl.kernel,
        mesh=mesh,
        compiler_params=pltpu.CompilerParams(use_tc_tiling_on_sc=False),
        out_type=jax.ShapeDtypeStruct((nslabs, NP, F), jnp.float32),
        scratch_types=[
            pltpu.VMEM((NCH, C), jnp.int32),
            pltpu.VMEM((C, F), jnp.float32),
            pltpu.VMEM((C, F), jnp.float32),
            pltpu.VMEM((C, F), jnp.float32),
            pltpu.VMEM((C, F), jnp.float32),
            pltpu.VMEM((NB, C), jnp.int32),
            pltpu.VMEM((NB, C), jnp.float32),
            pltpu.VMEM_SHARED((NP, F), jnp.float32),
            pltpu.SemaphoreType.DMA,
            pltpu.SemaphoreType.DMA,
            pltpu.SemaphoreType.DMA,
            pltpu.SemaphoreType.DMA,
            pltpu.SemaphoreType.DMA,
            pltpu.SemaphoreType.DMA,
            pltpu.SemaphoreType.DMA,
            pltpu.SemaphoreType.DMA,
        ],
    )
    def spmm(h2, colr, rowr, ewr, zeros, out, idx_v, rb0, rb1, rb2, rb3,
             row_cb, ew_cb, slab, sg0, sg1, sg2, sg3, ss0, ss1, ss2, ss3):
        cid = lax.axis_index("c")
        sid = lax.axis_index("s")
        stripe = pl.ds(sid * (NP // NS), NP // NS)
        rbufs = (rb0, rb1, rb2, rb3)
        sgs = (sg0, sg1, sg2, sg3)
        sss = (ss0, ss1, ss2, ss3)

        def issue(b, j):
            pltpu.async_copy(h2.at[idx_v.at[j]], rbufs[b], sgs[b])
            pltpu.async_copy(rowr.at[sid, j], row_cb.at[b], sgs[b])
            pltpu.async_copy(ewr.at[sid, j], ew_cb.at[b], sgs[b])

        def wait_in(b, j):
            pltpu.make_async_copy(h2.at[idx_v.at[j]], rbufs[b], sgs[b]).wait()
            pltpu.make_async_copy(rowr.at[sid, j], row_cb.at[b], sgs[b]).wait()
            pltpu.make_async_copy(ewr.at[sid, j], ew_cb.at[b], sgs[b]).wait()

        def scatter_start(b):
            pltpu.async_copy(rbufs[b], slab.at[row_cb.at[b]], sss[b], add=True)

        def scatter_wait(b):
            pltpu.make_async_copy(
                rbufs[b], slab.at[row_cb.at[b]], sss[b]).wait()

        for sl in range(spc):
            s = cid * spc + sl

            pltpu.sync_copy(zeros, slab.at[stripe, :])

            pltpu.sync_copy(colr.at[sid], idx_v)

            def idx_body(g, _):
                j = g // grp
                q = g - j * grp
                gs = pl.ds(q * 16, 16)
                idx_v[j, gs] = idx_v[j, gs] * nslabs + s
                return 0

            lax.fori_loop(0, NCH * grp, idx_body, 0)
            plsc.subcore_barrier()

            for b in range(NB - 1):
                issue(b, b)

            def quad_body(t, _):
                j0 = 4 * t
                for b in range(NB):
                    j = j0 + b
                    wait_in(b, j)

                    def scale_body(g, _):
                        w16 = ew_cb[b, pl.ds(g * 16, 16)]
                        for jj in range(16):
                            w = lax.broadcast(w16[jj], (16,))
                            e = g * 16 + jj
                            for f in range(F // 16):
                                fs = pl.ds(f * 16, 16)
                                rbufs[b][e, fs] = rbufs[b][e, fs] * w
                        return 0

                    lax.fori_loop(0, grp, scale_body, 0)
                    scatter_start(b)

                    p = (b + NB - 1) % NB
                    jp = j + NB - 1

                    @pl.when(jp >= NB)
                    def _():
                        scatter_wait(p)

                    @pl.when(jp < NCH)
                    def _():
                        issue(p, jp)

                return 0

            lax.fori_loop(0, NCH // 4, quad_body, 0)
            scatter_wait(NB - 1)
            plsc.subcore_barrier()

            pltpu.sync_copy(slab.at[stripe, :], out.at[s, stripe, :])

    return spmm


def _self_mm(h, w, b1, b2, bn=1000):
    n, din = h.shape
    hdim = w.shape[1]

    def body(h_ref, w_ref, b1_ref, b2_ref, o_ref):
        acc = jnp.dot(h_ref[...], w_ref[...], preferred_element_type=jnp.float32)
        o_ref[...] = acc + b1_ref[...] + b2_ref[...]

    return pl.pallas_call(
        body,
        grid=(n // bn,),
        in_specs=[
            pl.BlockSpec((bn, din), lambda i: (i, 0)),
            pl.BlockSpec((din, hdim), lambda i: (0, 0)),
            pl.BlockSpec((hdim,), lambda i: (0,)),
            pl.BlockSpec((hdim,), lambda i: (0,)),
        ],
        out_specs=pl.BlockSpec((bn, hdim), lambda i: (i, 0)),
        out_shape=jax.ShapeDtypeStruct((n, hdim), jnp.float32),
    )(h, w, b1, b2)


def _combine(hs, nei_t, wn_r, bn=1000):
    n, hdim = hs.shape
    nslabs = nei_t.shape[0]

    def body(hs_ref, nei_ref, wn_ref, o_ref):
        acc = hs_ref[...]
        for s in range(nslabs):
            acc = acc + jnp.dot(nei_ref[s], wn_ref[s],
                                preferred_element_type=jnp.float32)
        o_ref[...] = jnp.maximum(acc, 0.0)

    return pl.pallas_call(
        body,
        grid=(n // bn,),
        in_specs=[
            pl.BlockSpec((bn, hdim), lambda i: (i, 0)),
            pl.BlockSpec((nslabs, bn, F), lambda i: (0, i, 0)),
            pl.BlockSpec((nslabs, F, hdim), lambda i: (0, 0, 0)),
        ],
        out_specs=pl.BlockSpec((bn, hdim), lambda i: (i, 0)),
        out_shape=jax.ShapeDtypeStruct((n, hdim), jnp.float32),
    )(hs, nei_t, wn_r)


def _combine_final(hs, nei_t, wn_r, w_out, b_out, bn=1000):
    n, hdim = hs.shape
    nslabs = nei_t.shape[0]

    def body(hs_ref, nei_ref, wn_ref, wo_ref, bo_ref, o_ref):
        acc = hs_ref[...]
        for s in range(nslabs):
            acc = acc + jnp.dot(nei_ref[s], wn_ref[s],
                                preferred_element_type=jnp.float32)
        acc = jnp.maximum(acc, 0.0)
        o_ref[...] = jnp.dot(acc, wo_ref[...],
                             preferred_element_type=jnp.float32) + bo_ref[...]

    return pl.pallas_call(
        body,
        grid=(n // bn,),
        in_specs=[
            pl.BlockSpec((bn, hdim), lambda i: (i, 0)),
            pl.BlockSpec((nslabs, bn, F), lambda i: (0, i, 0)),
            pl.BlockSpec((nslabs, F, hdim), lambda i: (0, 0, 0)),
            pl.BlockSpec((hdim, 1), lambda i: (0, 0)),
            pl.BlockSpec((1,), lambda i: (0,)),
        ],
        out_specs=pl.BlockSpec((bn, 1), lambda i: (i, 0)),
        out_shape=jax.ShapeDtypeStruct((n, 1), jnp.float32),
    )(hs, nei_t, wn_r, w_out, b_out)


def kernel(x, edge_index, edge_weight, W_self_0, b_self_0, W_nei_0, b_nei_0,
           W_self_1, b_self_1, W_nei_1, b_nei_1, W_self_2, b_self_2, W_nei_2,
           b_nei_2, W_out, b_out):
    row = edge_index[0]
    col = edge_index[1]
    pad = E_PAD - E
    col_r = jnp.pad(col, (0, pad)).reshape(NS, NCH, C)
    row_r = jnp.pad(row, (0, pad)).reshape(NS, NCH, C)
    ew_r = jnp.pad(edge_weight, (0, pad)).reshape(NS, NCH, C)
    zeros = jnp.zeros((NP // NS, F), jnp.float32)

    params = [
        (W_self_0, b_self_0, W_nei_0, b_nei_0),
        (W_self_1, b_self_1, W_nei_1, b_nei_1),
        (W_self_2, b_self_2, W_nei_2, b_nei_2),
    ]
    h = x
    for k, (ws, bs, wn, bnei) in enumerate(params):
        din = h.shape[1]
        nslabs = din // F
        h2 = h.reshape(N * nslabs, F)
        nei_t = _make_spmm(din)(h2, col_r, row_r, ew_r, zeros)
        hs = _self_mm(h, ws, bs, bnei)
        wn_r = wn.reshape(nslabs, F, wn.shape[1])
        if k < 2:
            h = _combine(hs, nei_t, wn_r)
        else:
            out = _combine_final(hs, nei_t, wn_r, W_out, b_out)
    return out[:, 0]


# speedup vs baseline: 2.7037x; 1.3134x over previous
import functools

import jax
import jax.numpy as jnp
from jax import lax
from jax.experimental import pallas as pl
from jax.experimental.pallas import tpu as pltpu
from jax.experimental.pallas import tpu_sc as plsc

N = 10000
NP = 10240
E = 160000
F = 128
NC = 2
NS = 16
C = 16
EPS = E // NS
NCH = EPS // C


@functools.lru_cache(maxsize=None)
def _make_spmm(din):
    nslabs = din // F
    spc = nslabs // NC
    mesh = plsc.VectorSubcoreMesh(
        core_axis_name="c", subcore_axis_name="s", num_cores=NC, num_subcores=NS
    )
    grp = C // 16
    NB = 5

    @functools.partial(
        pl.kernel,
        mesh=mesh,
        compiler_params=pltpu.CompilerParams(use_tc_tiling_on_sc=False),
        out_type=jax.ShapeDtypeStruct((nslabs, NP, F), jnp.float32),
        scratch_types=[
            pltpu.VMEM((NCH, C), jnp.int32),
            pltpu.VMEM((C, F), jnp.float32),
            pltpu.VMEM((C, F), jnp.float32),
            pltpu.VMEM((C, F), jnp.float32),
            pltpu.VMEM((C, F), jnp.float32),
            pltpu.VMEM((C, F), jnp.float32),
            pltpu.VMEM((NB, C), jnp.int32),
            pltpu.VMEM((NB, C), jnp.float32),
            pltpu.VMEM_SHARED((NP, F), jnp.float32),
            pltpu.SemaphoreType.DMA,
            pltpu.SemaphoreType.DMA,
            pltpu.SemaphoreType.DMA,
            pltpu.SemaphoreType.DMA,
            pltpu.SemaphoreType.DMA,
            pltpu.SemaphoreType.DMA,
            pltpu.SemaphoreType.DMA,
            pltpu.SemaphoreType.DMA,
            pltpu.SemaphoreType.DMA,
            pltpu.SemaphoreType.DMA,
        ],
    )
    def spmm(h2, colr, rowr, ewr, zeros, out, idx_v, rb0, rb1, rb2, rb3, rb4,
             row_cb, ew_cb, slab, sg0, sg1, sg2, sg3, sg4,
             ss0, ss1, ss2, ss3, ss4):
        cid = lax.axis_index("c")
        sid = lax.axis_index("s")
        stripe = pl.ds(sid * (NP // NS), NP // NS)
        rbufs = (rb0, rb1, rb2, rb3, rb4)
        sgs = (sg0, sg1, sg2, sg3, sg4)
        sss = (ss0, ss1, ss2, ss3, ss4)

        def issue(b, j):
            pltpu.async_copy(h2.at[idx_v.at[j]], rbufs[b], sgs[b])
            pltpu.async_copy(rowr.at[sid, j], row_cb.at[b], sgs[b])
            pltpu.async_copy(ewr.at[sid, j], ew_cb.at[b], sgs[b])

        def wait_in(b, j):
            pltpu.make_async_copy(h2.at[idx_v.at[j]], rbufs[b], sgs[b]).wait()
            pltpu.make_async_copy(rowr.at[sid, j], row_cb.at[b], sgs[b]).wait()
            pltpu.make_async_copy(ewr.at[sid, j], ew_cb.at[b], sgs[b]).wait()

        def scatter_start(b):
            pltpu.async_copy(rbufs[b], slab.at[row_cb.at[b]], sss[b], add=True)

        def scatter_wait(b):
            pltpu.make_async_copy(
                rbufs[b], slab.at[row_cb.at[b]], sss[b]).wait()

        for sl in range(spc):
            s = cid * spc + sl

            pltpu.sync_copy(zeros, slab.at[stripe, :])

            pltpu.sync_copy(colr.at[sid], idx_v)

            def idx_body(g, _):
                j = g // grp
                q = g - j * grp
                gs = pl.ds(q * 16, 16)
                idx_v[j, gs] = idx_v[j, gs] * nslabs + s
                return 0

            lax.fori_loop(0, NCH * grp, idx_body, 0)
            plsc.subcore_barrier()

            for b in range(NB - 1):
                issue(b, b)

            def quad_body(t, _):
                j0 = NB * t
                for b in range(NB):
                    j = j0 + b
                    wait_in(b, j)

                    def scale_body(g, _):
                        w16 = ew_cb[b, pl.ds(g * 16, 16)]
                        for jj in range(16):
                            w = lax.broadcast(w16[jj], (16,))
                            e = g * 16 + jj
                            for f in range(F // 16):
                                fs = pl.ds(f * 16, 16)
                                rbufs[b][e, fs] = rbufs[b][e, fs] * w
                        return 0

                    lax.fori_loop(0, grp, scale_body, 0)
                    scatter_start(b)

                    p = (b + NB - 1) % NB
                    jp = j + NB - 1

                    @pl.when(jp >= NB)
                    def _():
                        scatter_wait(p)

                    @pl.when(jp < NCH)
                    def _():
                        issue(p, jp)

                return 0

            lax.fori_loop(0, NCH // NB, quad_body, 0)
            scatter_wait(NB - 1)
            plsc.subcore_barrier()

            pltpu.sync_copy(slab.at[stripe, :], out.at[s, stripe, :])

    return spmm


def _self_mm(h, w, b1, b2, bn=1000):
    n, din = h.shape
    hdim = w.shape[1]

    def body(h_ref, w_ref, b1_ref, b2_ref, o_ref):
        acc = jnp.dot(h_ref[...], w_ref[...], preferred_element_type=jnp.float32)
        o_ref[...] = acc + b1_ref[...] + b2_ref[...]

    return pl.pallas_call(
        body,
        grid=(n // bn,),
        in_specs=[
            pl.BlockSpec((bn, din), lambda i: (i, 0)),
            pl.BlockSpec((din, hdim), lambda i: (0, 0)),
            pl.BlockSpec((hdim,), lambda i: (0,)),
            pl.BlockSpec((hdim,), lambda i: (0,)),
        ],
        out_specs=pl.BlockSpec((bn, hdim), lambda i: (i, 0)),
        out_shape=jax.ShapeDtypeStruct((n, hdim), jnp.float32),
    )(h, w, b1, b2)


def _combine(hs, nei_t, wn_r, bn=1000):
    n, hdim = hs.shape
    nslabs = nei_t.shape[0]

    def body(hs_ref, nei_ref, wn_ref, o_ref):
        acc = hs_ref[...]
        for s in range(nslabs):
            acc = acc + jnp.dot(nei_ref[s], wn_ref[s],
                                preferred_element_type=jnp.float32)
        o_ref[...] = jnp.maximum(acc, 0.0)

    return pl.pallas_call(
        body,
        grid=(n // bn,),
        in_specs=[
            pl.BlockSpec((bn, hdim), lambda i: (i, 0)),
            pl.BlockSpec((nslabs, bn, F), lambda i: (0, i, 0)),
            pl.BlockSpec((nslabs, F, hdim), lambda i: (0, 0, 0)),
        ],
        out_specs=pl.BlockSpec((bn, hdim), lambda i: (i, 0)),
        out_shape=jax.ShapeDtypeStruct((n, hdim), jnp.float32),
    )(hs, nei_t, wn_r)


def _combine_final(hs, nei_t, wn_r, w_out, b_out, bn=1000):
    n, hdim = hs.shape
    nslabs = nei_t.shape[0]

    def body(hs_ref, nei_ref, wn_ref, wo_ref, bo_ref, o_ref):
        acc = hs_ref[...]
        for s in range(nslabs):
            acc = acc + jnp.dot(nei_ref[s], wn_ref[s],
                                preferred_element_type=jnp.float32)
        acc = jnp.maximum(acc, 0.0)
        o_ref[...] = jnp.dot(acc, wo_ref[...],
                             preferred_element_type=jnp.float32) + bo_ref[...]

    return pl.pallas_call(
        body,
        grid=(n // bn,),
        in_specs=[
            pl.BlockSpec((bn, hdim), lambda i: (i, 0)),
            pl.BlockSpec((nslabs, bn, F), lambda i: (0, i, 0)),
            pl.BlockSpec((nslabs, F, hdim), lambda i: (0, 0, 0)),
            pl.BlockSpec((hdim, 1), lambda i: (0, 0)),
            pl.BlockSpec((1,), lambda i: (0,)),
        ],
        out_specs=pl.BlockSpec((bn, 1), lambda i: (i, 0)),
        out_shape=jax.ShapeDtypeStruct((n, 1), jnp.float32),
    )(hs, nei_t, wn_r, w_out, b_out)


def kernel(x, edge_index, edge_weight, W_self_0, b_self_0, W_nei_0, b_nei_0,
           W_self_1, b_self_1, W_nei_1, b_nei_1, W_self_2, b_self_2, W_nei_2,
           b_nei_2, W_out, b_out):
    row = edge_index[0]
    col = edge_index[1]
    col_r = col.reshape(NS, NCH, C)
    row_r = row.reshape(NS, NCH, C)
    ew_r = edge_weight.reshape(NS, NCH, C)
    zeros = jnp.zeros((NP // NS, F), jnp.float32)

    params = [
        (W_self_0, b_self_0, W_nei_0, b_nei_0),
        (W_self_1, b_self_1, W_nei_1, b_nei_1),
        (W_self_2, b_self_2, W_nei_2, b_nei_2),
    ]
    h = x
    for k, (ws, bs, wn, bnei) in enumerate(params):
        din = h.shape[1]
        nslabs = din // F
        h2 = h.reshape(N * nslabs, F)
        nei_t = _make_spmm(din)(h2, col_r, row_r, ew_r, zeros)
        hs = _self_mm(h, ws, bs, bnei)
        wn_r = wn.reshape(nslabs, F, wn.shape[1])
        if k < 2:
            h = _combine(hs, nei_t, wn_r)
        else:
            out = _combine_final(hs, nei_t, wn_r, W_out, b_out)
    return out[:, 0]


# speedup vs baseline: 2.7373x; 1.0124x over previous
import functools

import jax
import jax.numpy as jnp
from jax import lax
from jax.experimental import pallas as pl
from jax.experimental.pallas import tpu as pltpu
from jax.experimental.pallas import tpu_sc as plsc

N = 10000
NP = 10240
E = 160000
F = 128
NC = 2
NS = 16
C = 16
EPS = E // NS
NCH = EPS // C


@functools.lru_cache(maxsize=None)
def _make_spmm(din):
    nslabs = din // F
    spc = nslabs // NC
    mesh = plsc.VectorSubcoreMesh(
        core_axis_name="c", subcore_axis_name="s", num_cores=NC, num_subcores=NS
    )
    grp = C // 16
    NB = 5

    @functools.partial(
        pl.kernel,
        mesh=mesh,
        compiler_params=pltpu.CompilerParams(use_tc_tiling_on_sc=False),
        out_type=jax.ShapeDtypeStruct((nslabs, NP, F), jnp.float32),
        scratch_types=[
            pltpu.VMEM((NCH, C), jnp.int32),
            pltpu.VMEM((C, F), jnp.float32),
            pltpu.VMEM((C, F), jnp.float32),
            pltpu.VMEM((C, F), jnp.float32),
            pltpu.VMEM((C, F), jnp.float32),
            pltpu.VMEM((C, F), jnp.float32),
            pltpu.VMEM((NB, C), jnp.int32),
            pltpu.VMEM((NB, C), jnp.float32),
            pltpu.VMEM_SHARED((NP, F), jnp.float32),
            pltpu.SemaphoreType.DMA,
            pltpu.SemaphoreType.DMA,
            pltpu.SemaphoreType.DMA,
            pltpu.SemaphoreType.DMA,
            pltpu.SemaphoreType.DMA,
            pltpu.SemaphoreType.DMA,
            pltpu.SemaphoreType.DMA,
            pltpu.SemaphoreType.DMA,
            pltpu.SemaphoreType.DMA,
            pltpu.SemaphoreType.DMA,
        ],
    )
    def spmm(h2, colr, rowr, ewr, zeros, out, idx_v, rb0, rb1, rb2, rb3, rb4,
             row_cb, ew_cb, slab, sg0, sg1, sg2, sg3, sg4,
             ss0, ss1, ss2, ss3, ss4):
        cid = lax.axis_index("c")
        sid = lax.axis_index("s")
        stripe = pl.ds(sid * (NP // NS), NP // NS)
        rbufs = (rb0, rb1, rb2, rb3, rb4)
        sgs = (sg0, sg1, sg2, sg3, sg4)
        sss = (ss0, ss1, ss2, ss3, ss4)

        def issue(b, j):
            pltpu.async_copy(h2.at[idx_v.at[j]], rbufs[b], sgs[b])
            pltpu.async_copy(rowr.at[sid, j], row_cb.at[b], sgs[b])
            pltpu.async_copy(ewr.at[sid, j], ew_cb.at[b], sgs[b])

        def wait_in(b, j):
            pltpu.make_async_copy(h2.at[idx_v.at[j]], rbufs[b], sgs[b]).wait()
            pltpu.make_async_copy(rowr.at[sid, j], row_cb.at[b], sgs[b]).wait()
            pltpu.make_async_copy(ewr.at[sid, j], ew_cb.at[b], sgs[b]).wait()

        def scatter_start(b):
            pltpu.async_copy(rbufs[b], slab.at[row_cb.at[b]], sss[b], add=True)

        def scatter_wait(b):
            pltpu.make_async_copy(
                rbufs[b], slab.at[row_cb.at[b]], sss[b]).wait()

        for sl in range(spc):
            s = cid * spc + sl

            pltpu.sync_copy(zeros, slab.at[stripe, :])

            pltpu.sync_copy(colr.at[sid], idx_v)

            def idx_body(g, _):
                j = g // grp
                q = g - j * grp
                gs = pl.ds(q * 16, 16)
                idx_v[j, gs] = idx_v[j, gs] * nslabs + s
                return 0

            lax.fori_loop(0, NCH * grp, idx_body, 0)
            plsc.subcore_barrier()

            for b in range(NB - 1):
                issue(b, b)

            def quad_body(t, _):
                j0 = NB * t
                for b in range(NB):
                    j = j0 + b
                    wait_in(b, j)

                    def scale_body(g, _):
                        w16 = ew_cb[b, pl.ds(g * 16, 16)]
                        for jj in range(16):
                            w = lax.broadcast(w16[jj], (16,))
                            e = g * 16 + jj
                            for f in range(F // 16):
                                fs = pl.ds(f * 16, 16)
                                rbufs[b][e, fs] = rbufs[b][e, fs] * w
                        return 0

                    lax.fori_loop(0, grp, scale_body, 0)
                    scatter_start(b)

                    p = (b + NB - 1) % NB
                    jp = j + NB - 1

                    @pl.when(jp >= NB)
                    def _():
                        scatter_wait(p)

                    @pl.when(jp < NCH)
                    def _():
                        issue(p, jp)

                return 0

            lax.fori_loop(0, NCH // NB, quad_body, 0)
            scatter_wait(NB - 1)
            plsc.subcore_barrier()

            pltpu.sync_copy(slab.at[stripe, :], out.at[s, stripe, :])

    return spmm


def _layer(h, w, b1, b2, nei_t, wn_r, bn=1000):
    n, din = h.shape
    hdim = w.shape[1]
    nslabs = nei_t.shape[0]

    def body(h_ref, w_ref, b1_ref, b2_ref, nei_ref, wn_ref, o_ref):
        acc = jnp.dot(h_ref[...], w_ref[...], preferred_element_type=jnp.float32)
        for s in range(nslabs):
            acc = acc + jnp.dot(nei_ref[s], wn_ref[s],
                                preferred_element_type=jnp.float32)
        o_ref[...] = jnp.maximum(acc + b1_ref[...] + b2_ref[...], 0.0)

    return pl.pallas_call(
        body,
        grid=(n // bn,),
        in_specs=[
            pl.BlockSpec((bn, din), lambda i: (i, 0)),
            pl.BlockSpec((din, hdim), lambda i: (0, 0)),
            pl.BlockSpec((hdim,), lambda i: (0,)),
            pl.BlockSpec((hdim,), lambda i: (0,)),
            pl.BlockSpec((nslabs, bn, F), lambda i: (0, i, 0)),
            pl.BlockSpec((nslabs, F, hdim), lambda i: (0, 0, 0)),
        ],
        out_specs=pl.BlockSpec((bn, hdim), lambda i: (i, 0)),
        out_shape=jax.ShapeDtypeStruct((n, hdim), jnp.float32),
    )(h, w, b1, b2, nei_t, wn_r)


def _layer_final(h, w, b1, b2, nei_t, wn_r, w_out, b_out, bn=1000):
    n, din = h.shape
    hdim = w.shape[1]
    nslabs = nei_t.shape[0]

    def body(h_ref, w_ref, b1_ref, b2_ref, nei_ref, wn_ref, wo_ref, bo_ref,
             o_ref):
        acc = jnp.dot(h_ref[...], w_ref[...], preferred_element_type=jnp.float32)
        for s in range(nslabs):
            acc = acc + jnp.dot(nei_ref[s], wn_ref[s],
                                preferred_element_type=jnp.float32)
        acc = jnp.maximum(acc + b1_ref[...] + b2_ref[...], 0.0)
        o_ref[...] = jnp.dot(acc, wo_ref[...],
                             preferred_element_type=jnp.float32) + bo_ref[...]

    return pl.pallas_call(
        body,
        grid=(n // bn,),
        in_specs=[
            pl.BlockSpec((bn, din), lambda i: (i, 0)),
            pl.BlockSpec((din, hdim), lambda i: (0, 0)),
            pl.BlockSpec((hdim,), lambda i: (0,)),
            pl.BlockSpec((hdim,), lambda i: (0,)),
            pl.BlockSpec((nslabs, bn, F), lambda i: (0, i, 0)),
            pl.BlockSpec((nslabs, F, hdim), lambda i: (0, 0, 0)),
            pl.BlockSpec((hdim, 1), lambda i: (0, 0)),
            pl.BlockSpec((1,), lambda i: (0,)),
        ],
        out_specs=pl.BlockSpec((bn, 1), lambda i: (i, 0)),
        out_shape=jax.ShapeDtypeStruct((n, 1), jnp.float32),
    )(h, w, b1, b2, nei_t, wn_r, w_out, b_out)


def kernel(x, edge_index, edge_weight, W_self_0, b_self_0, W_nei_0, b_nei_0,
           W_self_1, b_self_1, W_nei_1, b_nei_1, W_self_2, b_self_2, W_nei_2,
           b_nei_2, W_out, b_out):
    row = edge_index[0]
    col = edge_index[1]
    col_r = col.reshape(NS, NCH, C)
    row_r = row.reshape(NS, NCH, C)
    ew_r = edge_weight.reshape(NS, NCH, C)
    zeros = jnp.zeros((NP // NS, F), jnp.float32)

    params = [
        (W_self_0, b_self_0, W_nei_0, b_nei_0),
        (W_self_1, b_self_1, W_nei_1, b_nei_1),
        (W_self_2, b_self_2, W_nei_2, b_nei_2),
    ]
    h = x
    for k, (ws, bs, wn, bnei) in enumerate(params):
        din = h.shape[1]
        nslabs = din // F
        h2 = h.reshape(N * nslabs, F)
        nei_t = _make_spmm(din)(h2, col_r, row_r, ew_r, zeros)
        wn_r = wn.reshape(nslabs, F, wn.shape[1])
        if k < 2:
            h = _layer(h, ws, bs, bnei, nei_t, wn_r)
        else:
            out = _layer_final(h, ws, bs, bnei, nei_t, wn_r, W_out, b_out)
    return out[:, 0]
